# Initial kernel scaffold; baseline (speedup 1.0000x reference)
import jax
import jax.numpy as jnp
from jax.experimental import pallas as pl


def kernel(x, table):
    raise NotImplementedError("write your pallas kernel here")


# speedup vs baseline: 1.8433x; 1.8433x over previous
import jax
import jax.numpy as jnp
from jax import lax
from jax.experimental import pallas as pl
from jax.experimental.pallas import tpu as pltpu
from jax.experimental.pallas import tpu_sc as plsc

BATCH = 16384
HIST = 50
D = 64
N = BATCH * HIST

_info = plsc.get_sparse_core_info()
NC, NS = _info.num_cores, _info.num_subcores
NW = NC * NS
PER_W = N // NW
CHUNK = 800
NCHUNK = PER_W // CHUNK
GROUPS = CHUNK // 16


def _sc_body(idx_hbm, tab_hbm, out_hbm, idx_v, tab_v, out_v, sem):
    wid = lax.axis_index("s") * NC + lax.axis_index("c")
    pltpu.sync_copy(idx_hbm.at[wid], idx_v)
    pltpu.sync_copy(tab_hbm, tab_v)
    base = wid * PER_W
    iota = lax.iota(jnp.int32, 16)
    row_off = iota * D

    def group(g, c):
        xv = idx_v[pl.ds(c * CHUNK + g * 16, 16)]
        src = xv * D
        dst = g * (16 * D) + row_off
        for k in range(D):
            v = plsc.load_gather(tab_v, [src + k])
            plsc.store_scatter(out_v, [dst + k], v)
        return c

    def chunk(c, carry):
        lax.fori_loop(0, GROUPS, group, c)
        pltpu.async_copy(
            out_v, out_hbm.at[pl.ds((base + c * CHUNK) * D, CHUNK * D)], sem
        ).wait()
        return carry

    lax.fori_loop(0, NCHUNK, chunk, 0)


@jax.jit
def kernel(x, table):
    idx = x.reshape(NW, PER_W)
    out = pl.kernel(
        _sc_body,
        out_type=jax.ShapeDtypeStruct((N * D,), jnp.float32),
        mesh=plsc.VectorSubcoreMesh(core_axis_name="c", subcore_axis_name="s"),
        compiler_params=pltpu.CompilerParams(needs_layout_passes=False),
        scratch_types=[
            pltpu.VMEM((PER_W,), jnp.int32),
            pltpu.VMEM((2 * D,), jnp.float32),
            pltpu.VMEM((CHUNK * D,), jnp.float32),
            pltpu.SemaphoreType.DMA,
        ],
    )(idx, table.reshape(2 * D))
    return out.reshape(BATCH, HIST, D)
